# baseline (speedup 1.0000x reference)
import functools

import jax
import jax.numpy as jnp
from jax import lax
from jax.experimental import pallas as pl
from jax.experimental.pallas import tpu as pltpu
from jax.experimental.pallas import tpu_sc as plsc

N_NODES = 10000
N_EDGES = 320000
D = 128
DH = 64
NS = 16
K = 50
CPS = N_EDGES // NS // K
SBC = 16
NSB = CPS // SBC
FIN_CHUNK = 40
N_FIN = N_NODES // FIN_CHUNK
FIN_PER_SUB = (N_FIN + NS - 1) // NS
MSG_EPS = 1e-7
NODE_BLK = 1000
EDGE_BLK = 1000


def _node_enc_body(x_ref, w_ref, b_ref, o_ref):
    h = jnp.dot(x_ref[...], w_ref[...], preferred_element_type=jnp.float32)
    h = h + b_ref[...]
    o_ref[0] = h[:, :DH]
    o_ref[1] = h[:, DH:]


def _edge_enc_body(a_ref, w_ref, b_ref, o_ref):
    ea = jnp.dot(a_ref[...], w_ref[...], preferred_element_type=jnp.float32)
    ea = ea + b_ref[...]
    o_ref[0] = ea[:, :DH]
    o_ref[1] = ea[:, DH:]


def _mlp_body(agg_ref, h_ref, w1_ref, b1_ref, w2_ref, b2_ref, o_ref):
    u = jnp.concatenate([agg_ref[0] + h_ref[0], agg_ref[1] + h_ref[1]], axis=-1)
    p = jnp.dot(u, w1_ref[...], preferred_element_type=jnp.float32) + b1_ref[...]
    p = p / jnp.sqrt(1.0 + 1e-5)
    p = jnp.maximum(p, 0.0)
    y = jnp.dot(p, w2_ref[...], preferred_element_type=jnp.float32) + b2_ref[...]
    y = jnp.maximum(y, 0.0)
    o_ref[0] = y[:, :DH]
    o_ref[1] = y[:, DH:]


def _final_body(h_ref, wd0_ref, bd0_ref, wd1_ref, bd1_ref, o_ref):
    h2 = jnp.concatenate([h_ref[0], h_ref[1]], axis=-1)
    z = jnp.dot(h2, wd0_ref[...], preferred_element_type=jnp.float32) + bd0_ref[...]
    z = jnp.maximum(z, 0.0)
    z2 = jnp.dot(z, wd1_ref[...], preferred_element_type=jnp.float32) + bd1_ref[...]
    z2 = jnp.maximum(z2, 0.0)
    a = z2[:, 0:1]
    b = z2[:, 1:2]
    o_ref[...] = jnp.where(b > a, 1.0, 0.0)


def _full(shape):
    return pl.BlockSpec(shape, lambda i: tuple(0 for _ in shape))


def _node_encode(x, w, b):
    return pl.pallas_call(
        _node_enc_body,
        grid=(N_NODES // NODE_BLK,),
        in_specs=[
            pl.BlockSpec((NODE_BLK, D), lambda i: (i, 0)),
            _full((D, D)),
            _full((1, D)),
        ],
        out_specs=pl.BlockSpec((2, NODE_BLK, DH), lambda i: (0, i, 0)),
        out_shape=jax.ShapeDtypeStruct((2, N_NODES, DH), jnp.float32),
    )(x, w, b)


def _edge_encode(a, w, b):
    return pl.pallas_call(
        _edge_enc_body,
        grid=(N_EDGES // EDGE_BLK,),
        in_specs=[
            pl.BlockSpec((EDGE_BLK, 16), lambda i: (i, 0)),
            _full((16, D)),
            _full((1, D)),
        ],
        out_specs=pl.BlockSpec((2, EDGE_BLK, DH), lambda i: (0, i, 0)),
        out_shape=jax.ShapeDtypeStruct((2, N_EDGES, DH), jnp.float32),
    )(a, w, b)


def _mlp(agg, h, w1, b1, w2, b2):
    return pl.pallas_call(
        _mlp_body,
        grid=(N_NODES // NODE_BLK,),
        in_specs=[
            pl.BlockSpec((2, NODE_BLK, DH), lambda i: (0, i, 0)),
            pl.BlockSpec((2, NODE_BLK, DH), lambda i: (0, i, 0)),
            _full((D, 2 * D)),
            _full((1, 2 * D)),
            _full((2 * D, D)),
            _full((1, D)),
        ],
        out_specs=pl.BlockSpec((2, NODE_BLK, DH), lambda i: (0, i, 0)),
        out_shape=jax.ShapeDtypeStruct((2, N_NODES, DH), jnp.float32),
    )(agg, h, w1, b1, w2, b2)


def _final(h, wd0, bd0, wd1p, bd1p):
    return pl.pallas_call(
        _final_body,
        grid=(N_NODES // NODE_BLK,),
        in_specs=[
            pl.BlockSpec((2, NODE_BLK, DH), lambda i: (0, i, 0)),
            _full((D, D)),
            _full((1, D)),
            _full((D, D)),
            _full((1, D)),
        ],
        out_specs=pl.BlockSpec((NODE_BLK, 1), lambda i: (i, 0)),
        out_shape=jax.ShapeDtypeStruct((N_NODES, 1), jnp.float32),
    )(h, wd0, bd0, wd1p, bd1p)


def _sc_layer_body(h_hbm, ea_hbm, src_hbm, dst_hbm, t_hbm, out_hbm,
                   tv, ias, iad, ibs, ibd, g0, g1, e0, e1, buf0, buf1, acc,
                   sg0, sg1, se0, se1, ss0, ss1):
    c = lax.axis_index("c")
    s = lax.axis_index("s")
    gv_ = (g0, g1)
    ev_ = (e0, e1)
    bv_ = (buf0, buf1)
    sg_ = (sg0, sg1)
    se_ = (se0, se1)
    ss_ = (ss0, ss1)

    @pl.loop(0, FIN_CHUNK)
    def _zero_rows(r):
        for j in range(2 * DH // 16):
            buf0[r, pl.ds(j * 16, 16)] = jnp.zeros((16,), jnp.float32)

    for i in range(FIN_PER_SUB):
        chunk = s + NS * i

        @pl.when(chunk < N_FIN)
        def _zero_acc():
            pltpu.sync_copy(buf0.at[pl.ds(0, FIN_CHUNK)],
                            acc.at[pl.ds(chunk * FIN_CHUNK, FIN_CHUNK)])

    pltpu.sync_copy(t_hbm, tv)
    plsc.subcore_barrier()

    t = tv[...]
    base = s * CPS
    cbase = c * (N_EDGES // K) + base

    def _issue_g(idx_row, p):
        pltpu.async_copy(h_hbm.at[idx_row], gv_[p], sg_[p])

    def _issue_e(r, p):
        pltpu.async_copy(ea_hbm.at[r], ev_[p], se_[p])

    def _wait_g(p):
        pltpu.make_async_copy(h_hbm.at[ias.at[0]], gv_[p], sg_[p]).wait()

    def _wait_e(p):
        pltpu.make_async_copy(ea_hbm.at[cbase], ev_[p], se_[p]).wait()

    def _wait_s(p):
        pltpu.make_async_copy(bv_[p], acc.at[iad.at[0]], ss_[p]).wait()

    pltpu.sync_copy(src_hbm.at[pl.ds(cbase, 8)], ias)
    pltpu.sync_copy(dst_hbm.at[pl.ds(base, 8)], iad)
    _issue_g(ias.at[0], 0)
    _issue_e(cbase, 0)
    _issue_g(ias.at[1], 1)
    _issue_e(cbase + 1, 1)

    @pl.loop(0, NSB)
    def _superblock(u):
        r0 = base + u * SBC
        rc0 = cbase + u * SBC

        for b in range(SBC):
            p = b % 2
            gv, ev, bv = gv_[p], ev_[p], bv_[p]
            j = u * SBC + b

            if b < 2:
                @pl.when(u > 0)
                def _ws():
                    _wait_s(p)
            else:
                _wait_s(p)

            if b == 2:
                pltpu.sync_copy(src_hbm.at[pl.ds(rc0 + 8, 8)], ibs)
                pltpu.sync_copy(dst_hbm.at[pl.ds(r0 + 8, 8)], ibd)
            if b == 10:
                @pl.when(u < NSB - 1)
                def _ld():
                    pltpu.sync_copy(src_hbm.at[pl.ds(rc0 + 16, 8)], ias)
                    pltpu.sync_copy(dst_hbm.at[pl.ds(r0 + 16, 8)], iad)

            _wait_g(p)
            _wait_e(p)

            @plsc.parallel_loop(0, K, 1, unroll=2)
            def _row(k):
                for jj in range(DH // 16):
                    g = gv[k, pl.ds(jj * 16, 16)]
                    e = ev[k, pl.ds(jj * 16, 16)]
                    m = jnp.maximum(g + e, 0.0) + MSG_EPS
                    w = jnp.exp(m * t)
                    bv[k, pl.ds(jj * 16, 16)] = w
                    bv[k, pl.ds(DH + jj * 16, 16)] = m * w

            sidx = iad.at[b] if b < 8 else ibd.at[b - 8]
            pltpu.async_copy(bv, acc.at[sidx], ss_[p], add=True)

            if b < 6:
                gidx = ias.at[b + 2]
            elif b < 14:
                gidx = ibs.at[b - 6]
            else:
                gidx = ias.at[b - 14]
            if b >= 14:
                @pl.when(u < NSB - 1)
                def _pf():
                    _issue_g(gidx, p)
                    _issue_e(cbase + j + 2, p)
            else:
                _issue_g(gidx, p)
                _issue_e(cbase + j + 2, p)

    _wait_s(0)
    _wait_s(1)
    plsc.subcore_barrier()

    for i in range(FIN_PER_SUB):
        chunk = s + NS * i

        @pl.when(chunk < N_FIN)
        def _fin_chunk():
            row0 = chunk * FIN_CHUNK
            pltpu.sync_copy(acc.at[pl.ds(row0, FIN_CHUNK)],
                            buf0.at[pl.ds(0, FIN_CHUNK)])

            @pl.loop(0, FIN_CHUNK)
            def _fin_row(r):
                for j in range(DH // 16):
                    s0 = buf0[r, pl.ds(j * 16, 16)]
                    s1 = buf0[r, pl.ds(DH + j * 16, 16)]
                    e0[r, pl.ds(j * 16, 16)] = s1 / (s0 + 1e-16)

            pltpu.sync_copy(
                e0.at[pl.ds(0, FIN_CHUNK)],
                out_hbm.at[pl.ds(c * N_NODES + row0, FIN_CHUNK)])


_sc_layer = pl.kernel(
    _sc_layer_body,
    out_type=jax.ShapeDtypeStruct((2 * N_NODES, DH), jnp.float32),
    mesh=plsc.VectorSubcoreMesh(core_axis_name="c", subcore_axis_name="s"),
    compiler_params=pltpu.CompilerParams(use_tc_tiling_on_sc=False),
    scratch_types=[
        pltpu.VMEM((16,), jnp.float32),
        pltpu.VMEM((8, K), jnp.int32),
        pltpu.VMEM((8, K), jnp.int32),
        pltpu.VMEM((8, K), jnp.int32),
        pltpu.VMEM((8, K), jnp.int32),
        pltpu.VMEM((K, DH), jnp.float32),
        pltpu.VMEM((K, DH), jnp.float32),
        pltpu.VMEM((K, DH), jnp.float32),
        pltpu.VMEM((K, DH), jnp.float32),
        pltpu.VMEM((K, 2 * DH), jnp.float32),
        pltpu.VMEM((K, 2 * DH), jnp.float32),
        pltpu.VMEM_SHARED((N_NODES, 2 * DH), jnp.float32),
        pltpu.SemaphoreType.DMA,
        pltpu.SemaphoreType.DMA,
        pltpu.SemaphoreType.DMA,
        pltpu.SemaphoreType.DMA,
        pltpu.SemaphoreType.DMA,
        pltpu.SemaphoreType.DMA,
    ],
)


def kernel(x, edge_index, edge_attr, Wn, bn, We, be, t0,
           W1_0, b1_0, W2_0, b2_0, t1, W1_1, b1_1, W2_1, b2_1,
           Wd0, bd0, Wd1, bd1):
    src = edge_index[0]
    dst = edge_index[1]
    t0v = jnp.full((16,), t0, jnp.float32)
    t1v = jnp.full((16,), t1, jnp.float32)
    wd1p = jnp.zeros((D, D), jnp.float32).at[:, :2].set(Wd1)
    bd1p = jnp.zeros((1, D), jnp.float32).at[0, :2].set(bd1)

    h0 = _node_encode(x, Wn, bn.reshape(1, D))
    ea = _edge_encode(edge_attr, We, be.reshape(1, D))
    ea4 = ea.reshape(2 * N_EDGES // K, K, DH)
    src6 = jnp.concatenate([src, src + N_NODES]).reshape(2 * N_EDGES // K, K)
    dst6 = dst.reshape(N_EDGES // K, K)

    tvs = jnp.stack([t0v, t1v])
    w1s = jnp.stack([W1_0, W1_1])
    b1s = jnp.stack([b1_0.reshape(1, 2 * D), b1_1.reshape(1, 2 * D)])
    w2s = jnp.stack([W2_0, W2_1])
    b2s = jnp.stack([b2_0.reshape(1, D), b2_1.reshape(1, D)])

    nlayers = 1 + (t0 == t0).astype(jnp.int32)

    def _cond(carry):
        i, _ = carry
        return i < nlayers

    def _layer(carry):
        i, h = carry
        tv = lax.dynamic_index_in_dim(tvs, i, keepdims=False)
        w1 = lax.dynamic_index_in_dim(w1s, i, keepdims=False)
        b1 = lax.dynamic_index_in_dim(b1s, i, keepdims=False)
        w2 = lax.dynamic_index_in_dim(w2s, i, keepdims=False)
        b2 = lax.dynamic_index_in_dim(b2s, i, keepdims=False)
        agg = _sc_layer(h.reshape(2 * N_NODES, DH), ea4, src6, dst6, tv)
        h = _mlp(agg.reshape(2, N_NODES, DH), h, w1, b1, w2, b2)
        return (i + 1, h)

    _, h2 = lax.while_loop(_cond, _layer, (jnp.int32(0), h0))
    res = _final(h2, Wd0, bd0.reshape(1, D), wd1p, bd1p)
    return res.reshape(N_NODES)


# speedup vs baseline: 1.3246x; 1.3246x over previous
import functools

import jax
import jax.numpy as jnp
from jax import lax
from jax.experimental import pallas as pl
from jax.experimental.pallas import tpu as pltpu
from jax.experimental.pallas import tpu_sc as plsc

N_NODES = 10000
N_EDGES = 320000
D = 128
DH = 64
NS = 16
K = 50
CPS = N_EDGES // NS // K
SBC = 16
NSB = CPS // SBC
FIN_CHUNK = 40
N_FIN = N_NODES // FIN_CHUNK
FIN_PER_SUB = (N_FIN + NS - 1) // NS
MSG_EPS = 1e-7
NODE_BLK = 1000
EDGE_BLK = 2000


def _node_enc_body(x_ref, w_ref, b_ref, o_ref):
    h = jnp.dot(x_ref[...], w_ref[...], preferred_element_type=jnp.float32)
    o_ref[...] = h + b_ref[...]


def _edge_enc_body(a_ref, w_ref, b_ref, o_ref):
    ea = jnp.dot(a_ref[...], w_ref[...], preferred_element_type=jnp.float32)
    ea = ea + b_ref[...]
    ea2 = ea.reshape(EDGE_BLK // 2, 2, D)
    ev = ea2[:, 0, :]
    od = ea2[:, 1, :]
    o_ref[0] = jnp.concatenate([ev[:, :DH], od[:, :DH]], axis=-1)
    o_ref[1] = jnp.concatenate([ev[:, DH:], od[:, DH:]], axis=-1)


def _mlp_body(agg_ref, h_ref, w1_ref, b1_ref, w2_ref, b2_ref, o_ref):
    u = h_ref[...] + jnp.concatenate([agg_ref[0], agg_ref[1]], axis=-1)
    p = jnp.dot(u, w1_ref[...], preferred_element_type=jnp.float32) + b1_ref[...]
    p = p / jnp.sqrt(1.0 + 1e-5)
    p = jnp.maximum(p, 0.0)
    y = jnp.dot(p, w2_ref[...], preferred_element_type=jnp.float32) + b2_ref[...]
    o_ref[...] = jnp.maximum(y, 0.0)


def _final_body(h_ref, wd0_ref, bd0_ref, wd1_ref, bd1_ref, o_ref):
    h2 = h_ref[...]
    z = jnp.dot(h2, wd0_ref[...], preferred_element_type=jnp.float32) + bd0_ref[...]
    z = jnp.maximum(z, 0.0)
    z2 = jnp.dot(z, wd1_ref[...], preferred_element_type=jnp.float32) + bd1_ref[...]
    z2 = jnp.maximum(z2, 0.0)
    a = z2[:, 0:1]
    b = z2[:, 1:2]
    o_ref[...] = jnp.where(b > a, 1.0, 0.0)


def _full(shape):
    return pl.BlockSpec(shape, lambda i: tuple(0 for _ in shape))


def _node_encode(x, w, b):
    return pl.pallas_call(
        _node_enc_body,
        grid=(N_NODES // NODE_BLK,),
        in_specs=[
            pl.BlockSpec((NODE_BLK, D), lambda i: (i, 0)),
            _full((D, D)),
            _full((1, D)),
        ],
        out_specs=pl.BlockSpec((NODE_BLK, D), lambda i: (i, 0)),
        out_shape=jax.ShapeDtypeStruct((N_NODES, D), jnp.float32),
    )(x, w, b)


def _edge_encode(a, w, b):
    return pl.pallas_call(
        _edge_enc_body,
        grid=(N_EDGES // EDGE_BLK,),
        in_specs=[
            pl.BlockSpec((EDGE_BLK, 16), lambda i: (i, 0)),
            _full((16, D)),
            _full((1, D)),
        ],
        out_specs=pl.BlockSpec((2, EDGE_BLK // 2, D), lambda i: (0, i, 0)),
        out_shape=jax.ShapeDtypeStruct((2, N_EDGES // 2, D), jnp.float32),
    )(a, w, b)


def _mlp(agg, h, w1, b1, w2, b2):
    return pl.pallas_call(
        _mlp_body,
        grid=(N_NODES // NODE_BLK,),
        in_specs=[
            pl.BlockSpec((2, NODE_BLK, DH), lambda i: (0, i, 0)),
            pl.BlockSpec((NODE_BLK, D), lambda i: (i, 0)),
            _full((D, 2 * D)),
            _full((1, 2 * D)),
            _full((2 * D, D)),
            _full((1, D)),
        ],
        out_specs=pl.BlockSpec((NODE_BLK, D), lambda i: (i, 0)),
        out_shape=jax.ShapeDtypeStruct((N_NODES, D), jnp.float32),
    )(agg, h, w1, b1, w2, b2)


def _final(h, wd0, bd0, wd1p, bd1p):
    return pl.pallas_call(
        _final_body,
        grid=(N_NODES // NODE_BLK,),
        in_specs=[
            pl.BlockSpec((NODE_BLK, D), lambda i: (i, 0)),
            _full((D, D)),
            _full((1, D)),
            _full((D, D)),
            _full((1, D)),
        ],
        out_specs=pl.BlockSpec((NODE_BLK, 1), lambda i: (i, 0)),
        out_shape=jax.ShapeDtypeStruct((N_NODES, 1), jnp.float32),
    )(h, wd0, bd0, wd1p, bd1p)


def _sc_layer_body(h_hbm, ea_hbm, src_hbm, dst_hbm, t_hbm, out_hbm,
                   tv, ias, iad, ibs, ibd, g0, g1, e0, e1, buf0, buf1, acc,
                   sg0, sg1, se0, se1, ss0, ss1):
    c = lax.axis_index("c")
    s = lax.axis_index("s")
    gv_ = (g0, g1)
    ev_ = (e0, e1)
    bv_ = (buf0, buf1)
    sg_ = (sg0, sg1)
    se_ = (se0, se1)
    ss_ = (ss0, ss1)

    @pl.loop(0, FIN_CHUNK)
    def _zero_rows(r):
        for j in range(2 * DH // 16):
            buf0[r, pl.ds(j * 16, 16)] = jnp.zeros((16,), jnp.float32)

    for i in range(FIN_PER_SUB):
        chunk = s + NS * i

        @pl.when(chunk < N_FIN)
        def _zero_acc():
            pltpu.sync_copy(buf0.at[pl.ds(0, FIN_CHUNK)],
                            acc.at[pl.ds(chunk * FIN_CHUNK, FIN_CHUNK)])

    pltpu.sync_copy(t_hbm, tv)
    plsc.subcore_barrier()

    t = tv[...]
    cb = c * DH
    base = s * CPS
    ea_base = c * (N_EDGES // K) + base

    def _issue_g(idx_row, p):
        pltpu.async_copy(h_hbm.at[idx_row], gv_[p], sg_[p])

    def _issue_e(r, p):
        pltpu.async_copy(ea_hbm.at[r], ev_[p], se_[p])

    def _wait_g(p):
        pltpu.make_async_copy(h_hbm.at[ias.at[0]], gv_[p], sg_[p]).wait()

    def _wait_e(p):
        pltpu.make_async_copy(ea_hbm.at[ea_base], ev_[p], se_[p]).wait()

    def _wait_s(p):
        pltpu.make_async_copy(bv_[p], acc.at[iad.at[0]], ss_[p]).wait()

    pltpu.sync_copy(src_hbm.at[pl.ds(base, 8)], ias)
    pltpu.sync_copy(dst_hbm.at[pl.ds(base, 8)], iad)
    _issue_g(ias.at[0], 0)
    _issue_e(ea_base, 0)
    _issue_g(ias.at[1], 1)
    _issue_e(ea_base + 1, 1)

    @pl.loop(0, NSB)
    def _superblock(u):
        r0 = base + u * SBC

        for b in range(SBC):
            p = b % 2
            gv, ev, bv = gv_[p], ev_[p], bv_[p]
            j = u * SBC + b

            if b < 2:
                @pl.when(u > 0)
                def _ws():
                    _wait_s(p)
            else:
                _wait_s(p)

            if b == 2:
                pltpu.sync_copy(src_hbm.at[pl.ds(r0 + 8, 8)], ibs)
                pltpu.sync_copy(dst_hbm.at[pl.ds(r0 + 8, 8)], ibd)
            if b == 10:
                @pl.when(u < NSB - 1)
                def _ld():
                    pltpu.sync_copy(src_hbm.at[pl.ds(r0 + 16, 8)], ias)
                    pltpu.sync_copy(dst_hbm.at[pl.ds(r0 + 16, 8)], iad)

            _wait_g(p)
            _wait_e(p)

            @plsc.parallel_loop(0, K, 1, unroll=2)
            def _row(k):
                kh = k // 2
                ko = (k % 2) * DH
                for jj in range(DH // 16):
                    g = gv[k, pl.ds(cb + jj * 16, 16)]
                    e = ev[kh, pl.ds(ko + jj * 16, 16)]
                    m = jnp.maximum(g + e, 0.0) + MSG_EPS
                    w = jnp.exp(m * t)
                    bv[k, pl.ds(jj * 16, 16)] = w
                    bv[k, pl.ds(DH + jj * 16, 16)] = m * w

            sidx = iad.at[b] if b < 8 else ibd.at[b - 8]
            pltpu.async_copy(bv, acc.at[sidx], ss_[p], add=True)

            if b < 6:
                gidx = ias.at[b + 2]
            elif b < 14:
                gidx = ibs.at[b - 6]
            else:
                gidx = ias.at[b - 14]
            if b >= 14:
                @pl.when(u < NSB - 1)
                def _pf():
                    _issue_g(gidx, p)
                    _issue_e(ea_base + j + 2, p)
            else:
                _issue_g(gidx, p)
                _issue_e(ea_base + j + 2, p)

    _wait_s(0)
    _wait_s(1)
    plsc.subcore_barrier()

    for i in range(FIN_PER_SUB):
        chunk = s + NS * i

        @pl.when(chunk < N_FIN)
        def _fin_chunk():
            row0 = chunk * FIN_CHUNK
            pltpu.sync_copy(acc.at[pl.ds(row0, FIN_CHUNK)],
                            buf0.at[pl.ds(0, FIN_CHUNK)])

            @pl.loop(0, FIN_CHUNK)
            def _fin_row(r):
                rh = r // 2
                ro = (r % 2) * DH
                for j in range(DH // 16):
                    s0 = buf0[r, pl.ds(j * 16, 16)]
                    s1 = buf0[r, pl.ds(DH + j * 16, 16)]
                    e0[rh, pl.ds(ro + j * 16, 16)] = s1 / (s0 + 1e-16)

            pltpu.sync_copy(e0.at[pl.ds(0, FIN_CHUNK // 2)],
                            out_hbm.at[c * N_FIN + chunk])


_sc_layer = pl.kernel(
    _sc_layer_body,
    out_type=jax.ShapeDtypeStruct((2 * N_FIN, FIN_CHUNK // 2, D), jnp.float32),
    mesh=plsc.VectorSubcoreMesh(core_axis_name="c", subcore_axis_name="s"),
    scratch_types=[
        pltpu.VMEM((16,), jnp.float32),
        pltpu.VMEM((8, K), jnp.int32),
        pltpu.VMEM((8, K), jnp.int32),
        pltpu.VMEM((8, K), jnp.int32),
        pltpu.VMEM((8, K), jnp.int32),
        pltpu.VMEM((K, D), jnp.float32),
        pltpu.VMEM((K, D), jnp.float32),
        pltpu.VMEM((K // 2, 2 * DH), jnp.float32),
        pltpu.VMEM((K // 2, 2 * DH), jnp.float32),
        pltpu.VMEM((K, 2 * DH), jnp.float32),
        pltpu.VMEM((K, 2 * DH), jnp.float32),
        pltpu.VMEM_SHARED((N_NODES, 2 * DH), jnp.float32),
        pltpu.SemaphoreType.DMA,
        pltpu.SemaphoreType.DMA,
        pltpu.SemaphoreType.DMA,
        pltpu.SemaphoreType.DMA,
        pltpu.SemaphoreType.DMA,
        pltpu.SemaphoreType.DMA,
    ],
)


def kernel(x, edge_index, edge_attr, Wn, bn, We, be, t0,
           W1_0, b1_0, W2_0, b2_0, t1, W1_1, b1_1, W2_1, b2_1,
           Wd0, bd0, Wd1, bd1):
    src = edge_index[0]
    dst = edge_index[1]
    t0v = jnp.full((16,), t0, jnp.float32)
    t1v = jnp.full((16,), t1, jnp.float32)
    wd1p = jnp.zeros((D, D), jnp.float32).at[:, :2].set(Wd1)
    bd1p = jnp.zeros((1, D), jnp.float32).at[0, :2].set(bd1)

    h0 = _node_encode(x, Wn, bn.reshape(1, D))
    ea = _edge_encode(edge_attr, We, be.reshape(1, D))
    ea4 = ea.reshape(2 * N_EDGES // K, K // 2, 2 * DH)
    src6 = src.reshape(N_EDGES // K, K)
    dst6 = dst.reshape(N_EDGES // K, K)

    tvs = jnp.stack([t0v, t1v])
    w1s = jnp.stack([W1_0, W1_1])
    b1s = jnp.stack([b1_0.reshape(1, 2 * D), b1_1.reshape(1, 2 * D)])
    w2s = jnp.stack([W2_0, W2_1])
    b2s = jnp.stack([b2_0.reshape(1, D), b2_1.reshape(1, D)])

    nlayers = 1 + (t0 == t0).astype(jnp.int32)

    def _cond(carry):
        i, _ = carry
        return i < nlayers

    def _layer(carry):
        i, h = carry
        tv = lax.dynamic_index_in_dim(tvs, i, keepdims=False)
        w1 = lax.dynamic_index_in_dim(w1s, i, keepdims=False)
        b1 = lax.dynamic_index_in_dim(b1s, i, keepdims=False)
        w2 = lax.dynamic_index_in_dim(w2s, i, keepdims=False)
        b2 = lax.dynamic_index_in_dim(b2s, i, keepdims=False)
        agg = _sc_layer(h, ea4, src6, dst6, tv)
        h = _mlp(agg.reshape(2, N_NODES, DH), h, w1, b1, w2, b2)
        return (i + 1, h)

    _, h2 = lax.while_loop(_cond, _layer, (jnp.int32(0), h0))
    res = _final(h2, Wd0, bd0.reshape(1, D), wd1p, bd1p)
    return res.reshape(N_NODES)


# speedup vs baseline: 1.3910x; 1.0501x over previous
import functools

import jax
import jax.numpy as jnp
from jax import lax
from jax.experimental import pallas as pl
from jax.experimental.pallas import tpu as pltpu
from jax.experimental.pallas import tpu_sc as plsc

N_NODES = 10000
N_EDGES = 320000
D = 128
DH = 64
NS = 16
K = 50
CPS = N_EDGES // NS // K
SBC = 16
NSB = CPS // SBC
FIN_CHUNK = 40
N_FIN = N_NODES // FIN_CHUNK
FIN_PER_SUB = (N_FIN + NS - 1) // NS
MSG_EPS = 1e-7
NODE_BLK = 1000
EDGE_BLK = 2000


def _node_enc_body(x_ref, w_ref, b_ref, o_ref):
    h = jnp.dot(x_ref[...], w_ref[...], preferred_element_type=jnp.float32)
    o_ref[...] = h + b_ref[...]


def _edge_enc_body(a_ref, w_ref, b_ref, o_ref):
    ea = jnp.dot(a_ref[...], w_ref[...], preferred_element_type=jnp.float32)
    ea = ea + b_ref[...]
    ea2 = ea.reshape(EDGE_BLK // 2, 2, D)
    ev = ea2[:, 0, :]
    od = ea2[:, 1, :]
    o_ref[0] = jnp.concatenate([ev[:, :DH], od[:, :DH]], axis=-1)
    o_ref[1] = jnp.concatenate([ev[:, DH:], od[:, DH:]], axis=-1)


def _mlp_body(agg_ref, h_ref, w1_ref, b1_ref, w2_ref, b2_ref, o_ref):
    u = h_ref[...] + jnp.concatenate([agg_ref[0], agg_ref[1]], axis=-1)
    p = jnp.dot(u, w1_ref[...], preferred_element_type=jnp.float32) + b1_ref[...]
    p = p / jnp.sqrt(1.0 + 1e-5)
    p = jnp.maximum(p, 0.0)
    y = jnp.dot(p, w2_ref[...], preferred_element_type=jnp.float32) + b2_ref[...]
    o_ref[...] = jnp.maximum(y, 0.0)


def _final_body(h_ref, wd0_ref, bd0_ref, wd1_ref, bd1_ref, o_ref):
    h2 = h_ref[...]
    z = jnp.dot(h2, wd0_ref[...], preferred_element_type=jnp.float32) + bd0_ref[...]
    z = jnp.maximum(z, 0.0)
    z2 = jnp.dot(z, wd1_ref[...], preferred_element_type=jnp.float32) + bd1_ref[...]
    z2 = jnp.maximum(z2, 0.0)
    a = z2[:, 0:1]
    b = z2[:, 1:2]
    o_ref[...] = jnp.where(b > a, 1.0, 0.0)


def _full(shape):
    return pl.BlockSpec(shape, lambda i: tuple(0 for _ in shape))


def _node_encode(x, w, b):
    return pl.pallas_call(
        _node_enc_body,
        grid=(N_NODES // NODE_BLK,),
        in_specs=[
            pl.BlockSpec((NODE_BLK, D), lambda i: (i, 0)),
            _full((D, D)),
            _full((1, D)),
        ],
        out_specs=pl.BlockSpec((NODE_BLK, D), lambda i: (i, 0)),
        out_shape=jax.ShapeDtypeStruct((N_NODES, D), jnp.float32),
    )(x, w, b)


def _edge_encode(a, w, b):
    return pl.pallas_call(
        _edge_enc_body,
        grid=(N_EDGES // EDGE_BLK,),
        in_specs=[
            pl.BlockSpec((EDGE_BLK, 16), lambda i: (i, 0)),
            _full((16, D)),
            _full((1, D)),
        ],
        out_specs=pl.BlockSpec((2, EDGE_BLK // 2, D), lambda i: (0, i, 0)),
        out_shape=jax.ShapeDtypeStruct((2, N_EDGES // 2, D), jnp.float32),
    )(a, w, b)


def _mlp(agg, h, w1, b1, w2, b2):
    return pl.pallas_call(
        _mlp_body,
        grid=(N_NODES // NODE_BLK,),
        in_specs=[
            pl.BlockSpec((2, NODE_BLK, DH), lambda i: (0, i, 0)),
            pl.BlockSpec((NODE_BLK, D), lambda i: (i, 0)),
            _full((D, 2 * D)),
            _full((1, 2 * D)),
            _full((2 * D, D)),
            _full((1, D)),
        ],
        out_specs=pl.BlockSpec((NODE_BLK, D), lambda i: (i, 0)),
        out_shape=jax.ShapeDtypeStruct((N_NODES, D), jnp.float32),
    )(agg, h, w1, b1, w2, b2)


def _final(h, wd0, bd0, wd1p, bd1p):
    return pl.pallas_call(
        _final_body,
        grid=(N_NODES // NODE_BLK,),
        in_specs=[
            pl.BlockSpec((NODE_BLK, D), lambda i: (i, 0)),
            _full((D, D)),
            _full((1, D)),
            _full((D, D)),
            _full((1, D)),
        ],
        out_specs=pl.BlockSpec((NODE_BLK, 1), lambda i: (i, 0)),
        out_shape=jax.ShapeDtypeStruct((N_NODES, 1), jnp.float32),
    )(h, wd0, bd0, wd1p, bd1p)


def _sc_layer_body(h_hbm, ea_hbm, src_hbm, dst_hbm, t_hbm, out_hbm,
                   tv, ias, iad, ibs, ibd, g0, g1, e0, e1, buf0, buf1, acc,
                   sg0, sg1, se0, se1, ss0, ss1):
    c = lax.axis_index("c")
    s = lax.axis_index("s")
    gv_ = (g0, g1)
    ev_ = (e0, e1)
    bv_ = (buf0, buf1)
    sg_ = (sg0, sg1)
    ss_ = (ss0, ss1)
    si_ = (se0, se1)

    @pl.loop(0, FIN_CHUNK)
    def _zero_rows(r):
        for j in range(2 * DH // 16):
            buf0[r, pl.ds(j * 16, 16)] = jnp.zeros((16,), jnp.float32)

    for i in range(FIN_PER_SUB):
        chunk = s + NS * i

        @pl.when(chunk < N_FIN)
        def _zero_acc():
            pltpu.sync_copy(buf0.at[pl.ds(0, FIN_CHUNK)],
                            acc.at[pl.ds(chunk * FIN_CHUNK, FIN_CHUNK)])

    pltpu.sync_copy(t_hbm, tv)
    plsc.subcore_barrier()

    t = tv[...]
    cb = c * DH
    base = s * CPS
    ea_base = c * (N_EDGES // K) + base

    def _issue_g(idx_row, p):
        pltpu.async_copy(h_hbm.at[idx_row], gv_[p], sg_[p])

    def _issue_e(r, p):
        pltpu.async_copy(ea_hbm.at[r], ev_[p], sg_[p])

    def _wait_ge(p):
        pltpu.make_async_copy(h_hbm.at[ias.at[0]], gv_[p], sg_[p]).wait()
        pltpu.make_async_copy(ea_hbm.at[ea_base], ev_[p], sg_[p]).wait()

    def _wait_s(p):
        pltpu.make_async_copy(bv_[p], acc.at[iad.at[0]], ss_[p]).wait()

    pltpu.sync_copy(src_hbm.at[pl.ds(base, 8)], ias)
    pltpu.sync_copy(dst_hbm.at[pl.ds(base, 8)], iad)
    _issue_g(ias.at[0], 0)
    _issue_e(ea_base, 0)
    _issue_g(ias.at[1], 1)
    _issue_e(ea_base + 1, 1)

    @pl.loop(0, NSB)
    def _superblock(u):
        r0 = base + u * SBC

        for b in range(SBC):
            p = b % 2
            gv, ev, bv = gv_[p], ev_[p], bv_[p]
            j = u * SBC + b

            if b < 2:
                @pl.when(u > 0)
                def _ws():
                    _wait_s(p)
            else:
                _wait_s(p)

            if b == 2:
                pltpu.async_copy(src_hbm.at[pl.ds(r0 + 8, 8)], ibs, si_[1])
                pltpu.async_copy(dst_hbm.at[pl.ds(r0 + 8, 8)], ibd, si_[1])
            if b == 10:
                @pl.when(u < NSB - 1)
                def _ld():
                    pltpu.async_copy(src_hbm.at[pl.ds(r0 + 16, 8)], ias, si_[0])
                    pltpu.async_copy(dst_hbm.at[pl.ds(r0 + 16, 8)], iad, si_[0])
            if b == 6:
                pltpu.make_async_copy(src_hbm.at[pl.ds(r0, 8)], ibs, si_[1]).wait()
                pltpu.make_async_copy(dst_hbm.at[pl.ds(r0, 8)], ibd, si_[1]).wait()
            if b == 14:
                @pl.when(u < NSB - 1)
                def _wi():
                    pltpu.make_async_copy(
                        src_hbm.at[pl.ds(r0, 8)], ias, si_[0]).wait()
                    pltpu.make_async_copy(
                        dst_hbm.at[pl.ds(r0, 8)], iad, si_[0]).wait()

            _wait_ge(p)

            @plsc.parallel_loop(0, K, 1, unroll=2)
            def _row(k):
                kh = k // 2
                ko = (k % 2) * DH
                for jj in range(DH // 16):
                    g = gv[k, pl.ds(cb + jj * 16, 16)]
                    e = ev[kh, pl.ds(ko + jj * 16, 16)]
                    m = jnp.maximum(g + e, 0.0) + MSG_EPS
                    w = jnp.exp(m * t)
                    bv[k, pl.ds(jj * 16, 16)] = w
                    bv[k, pl.ds(DH + jj * 16, 16)] = m * w

            sidx = iad.at[b] if b < 8 else ibd.at[b - 8]
            pltpu.async_copy(bv, acc.at[sidx], ss_[p], add=True)

            if b < 6:
                gidx = ias.at[b + 2]
            elif b < 14:
                gidx = ibs.at[b - 6]
            else:
                gidx = ias.at[b - 14]
            if b >= 14:
                @pl.when(u < NSB - 1)
                def _pf():
                    _issue_g(gidx, p)
                    _issue_e(ea_base + j + 2, p)
            else:
                _issue_g(gidx, p)
                _issue_e(ea_base + j + 2, p)

    _wait_s(0)
    _wait_s(1)
    plsc.subcore_barrier()

    for i in range(FIN_PER_SUB):
        chunk = s + NS * i

        @pl.when(chunk < N_FIN)
        def _fin_chunk():
            row0 = chunk * FIN_CHUNK
            pltpu.sync_copy(acc.at[pl.ds(row0, FIN_CHUNK)],
                            buf0.at[pl.ds(0, FIN_CHUNK)])

            @pl.loop(0, FIN_CHUNK)
            def _fin_row(r):
                rh = r // 2
                ro = (r % 2) * DH
                for j in range(DH // 16):
                    s0 = buf0[r, pl.ds(j * 16, 16)]
                    s1 = buf0[r, pl.ds(DH + j * 16, 16)]
                    e0[rh, pl.ds(ro + j * 16, 16)] = s1 / (s0 + 1e-16)

            pltpu.sync_copy(e0.at[pl.ds(0, FIN_CHUNK // 2)],
                            out_hbm.at[c * N_FIN + chunk])


_sc_layer = pl.kernel(
    _sc_layer_body,
    out_type=jax.ShapeDtypeStruct((2 * N_FIN, FIN_CHUNK // 2, D), jnp.float32),
    mesh=plsc.VectorSubcoreMesh(core_axis_name="c", subcore_axis_name="s"),
    scratch_types=[
        pltpu.VMEM((16,), jnp.float32),
        pltpu.VMEM((8, K), jnp.int32),
        pltpu.VMEM((8, K), jnp.int32),
        pltpu.VMEM((8, K), jnp.int32),
        pltpu.VMEM((8, K), jnp.int32),
        pltpu.VMEM((K, D), jnp.float32),
        pltpu.VMEM((K, D), jnp.float32),
        pltpu.VMEM((K // 2, 2 * DH), jnp.float32),
        pltpu.VMEM((K // 2, 2 * DH), jnp.float32),
        pltpu.VMEM((K, 2 * DH), jnp.float32),
        pltpu.VMEM((K, 2 * DH), jnp.float32),
        pltpu.VMEM_SHARED((N_NODES, 2 * DH), jnp.float32),
        pltpu.SemaphoreType.DMA,
        pltpu.SemaphoreType.DMA,
        pltpu.SemaphoreType.DMA,
        pltpu.SemaphoreType.DMA,
        pltpu.SemaphoreType.DMA,
        pltpu.SemaphoreType.DMA,
    ],
)


def kernel(x, edge_index, edge_attr, Wn, bn, We, be, t0,
           W1_0, b1_0, W2_0, b2_0, t1, W1_1, b1_1, W2_1, b2_1,
           Wd0, bd0, Wd1, bd1):
    src = edge_index[0]
    dst = edge_index[1]
    t0v = jnp.full((16,), t0, jnp.float32)
    t1v = jnp.full((16,), t1, jnp.float32)
    wd1p = jnp.zeros((D, D), jnp.float32).at[:, :2].set(Wd1)
    bd1p = jnp.zeros((1, D), jnp.float32).at[0, :2].set(bd1)

    h0 = _node_encode(x, Wn, bn.reshape(1, D))
    ea = _edge_encode(edge_attr, We, be.reshape(1, D))
    ea4 = ea.reshape(2 * N_EDGES // K, K // 2, 2 * DH)
    src6 = src.reshape(N_EDGES // K, K)
    dst6 = dst.reshape(N_EDGES // K, K)

    tvs = jnp.stack([t0v, t1v])
    w1s = jnp.stack([W1_0, W1_1])
    b1s = jnp.stack([b1_0.reshape(1, 2 * D), b1_1.reshape(1, 2 * D)])
    w2s = jnp.stack([W2_0, W2_1])
    b2s = jnp.stack([b2_0.reshape(1, D), b2_1.reshape(1, D)])

    nlayers = 1 + (t0 == t0).astype(jnp.int32)

    def _cond(carry):
        i, _ = carry
        return i < nlayers

    def _layer(carry):
        i, h = carry
        tv = lax.dynamic_index_in_dim(tvs, i, keepdims=False)
        w1 = lax.dynamic_index_in_dim(w1s, i, keepdims=False)
        b1 = lax.dynamic_index_in_dim(b1s, i, keepdims=False)
        w2 = lax.dynamic_index_in_dim(w2s, i, keepdims=False)
        b2 = lax.dynamic_index_in_dim(b2s, i, keepdims=False)
        agg = _sc_layer(h, ea4, src6, dst6, tv)
        h = _mlp(agg.reshape(2, N_NODES, DH), h, w1, b1, w2, b2)
        return (i + 1, h)

    _, h2 = lax.while_loop(_cond, _layer, (jnp.int32(0), h0))
    res = _final(h2, Wd0, bd0.reshape(1, D), wd1p, bd1p)
    return res.reshape(N_NODES)
